# baseline (speedup 1.0000x reference)
import functools

import jax
import jax.numpy as jnp
from jax import lax
from jax.experimental import pallas as pl
from jax.experimental.pallas import tpu as pltpu
from jax.experimental.pallas import tpu_sc as plsc

_VOCAB = 1000000
_MOD = _VOCAB - 1
_D = 64
_N_OUT = 1024
_SEQ = 8192

_NC, _NS = 2, 16
_NW = _NC * _NS
_LANES = 16
_IDX_CHUNK = 128


def _make_sc_hash_gather(n_tok):
    b_per_w = n_tok // _NW
    n_vec = b_per_w // _LANES
    n_chunk = b_per_w // _IDX_CHUNK
    mesh = plsc.VectorSubcoreMesh(core_axis_name="c", subcore_axis_name="s")

    @functools.partial(
        pl.kernel,
        out_type=jax.ShapeDtypeStruct((n_tok, 2 * _D), jnp.float32),
        mesh=mesh,
        scratch_types=[
            pltpu.VMEM((b_per_w,), jnp.int32),
            pltpu.VMEM((b_per_w,), jnp.int32),
            pltpu.VMEM((n_chunk, _IDX_CHUNK), jnp.int32),
            pltpu.VMEM((b_per_w,), jnp.int32),
            pltpu.VMEM((b_per_w // 2, 2 * _D), jnp.float32),
            pltpu.SemaphoreType.DMA,
        ],
    )
    def sc_kernel(tok_hbm, tokp_hbm, table_hbm, out_hbm,
                  tbuf, pbuf, idx, par, rows, sem):
        wid = lax.axis_index("s") * _NC + lax.axis_index("c")
        base = wid * b_per_w
        srs = jnp.int32(1) - jnp.minimum(lax.rem(base, jnp.int32(_SEQ)),
                                         jnp.int32(1))

        pltpu.sync_copy(tok_hbm.at[pl.ds(base, b_per_w)], tbuf)
        pltpu.sync_copy(tokp_hbm.at[pl.ds(base, b_per_w)], pbuf)

        lanes = lax.iota(jnp.int32, _LANES)
        lane0 = jnp.int32(1) - jnp.minimum(lanes, jnp.int32(1))
        for j in range(n_vec):
            cur = tbuf[pl.ds(_LANES * j, _LANES)]
            prev = pbuf[pl.ds(_LANES * j, _LANES)]
            mixed = jnp.int32(36313) * cur ^ jnp.int32(27191) * prev
            r = lax.rem(mixed, jnp.int32(_MOD))
            r = r - (r >> 31) * jnp.int32(_MOD)
            if j == 0:
                m = lane0 * srs
                r = r + m * (jnp.int32(_MOD) - r)
            idx[j // 8, pl.ds((j % 8) * _LANES, _LANES)] = r >> 1
            par[pl.ds(_LANES * j, _LANES)] = (r & 1) * jnp.int32(_D)

        zeros = jnp.zeros((_LANES,), jnp.float32)
        half = b_per_w // 2
        for h in range(2):
            copies = [
                pltpu.async_copy(
                    table_hbm.at[idx.at[h * (n_chunk // 2) + c]],
                    rows.at[pl.ds(c * _IDX_CHUNK, _IDX_CHUNK), :],
                    sem,
                )
                for c in range(n_chunk // 2)
            ]
            for cp in copies:
                cp.wait()

            def zero_body(j, carry):
                pv = par[pl.ds(h * half + j * _LANES, _LANES)]
                for k in range(_LANES):
                    keep = pv[k]
                    kill = jnp.int32(_D) - keep
                    i = j * _LANES + k
                    for q in range(_D // _LANES):
                        rows[i, pl.ds(kill + q * _LANES, _LANES)] = zeros
                return carry

            lax.fori_loop(0, half // _LANES, zero_body, jnp.int32(0))
            pltpu.sync_copy(rows, out_hbm.at[pl.ds(base + h * half, half)])

    return sc_kernel


_TM = 512


def _tc_project(gathered, proj_t, scale):
    n_tok = gathered.shape[0]

    def body(scale_ref, g_ref, p_ref, o_ref):
        lhs = g_ref[:, : _D] + g_ref[:, _D:]
        o_ref[...] = (
            jnp.dot(lhs, p_ref[...], preferred_element_type=jnp.float32)
            * scale_ref[0]
        )

    return pl.pallas_call(
        body,
        grid=(n_tok // _TM,),
        in_specs=[
            pl.BlockSpec(memory_space=pltpu.SMEM),
            pl.BlockSpec((_TM, 2 * _D), lambda i: (i, 0)),
            pl.BlockSpec((_D, _N_OUT), lambda i: (0, 0)),
        ],
        out_specs=pl.BlockSpec((_TM, _N_OUT), lambda i: (i, 0)),
        out_shape=jax.ShapeDtypeStruct((n_tok, _N_OUT), jnp.float32),
    )(scale.reshape(1).astype(jnp.float32), gathered, proj_t)


def kernel(token_ids, embed_weight, proj_weight, scale):
    b, s = token_ids.shape
    tok2d = token_ids.astype(jnp.int32)
    tokp2d = jnp.concatenate([tok2d[:, :1], tok2d[:, :-1]], axis=1)
    tok = tok2d.reshape(-1)
    tokp = tokp2d.reshape(-1)
    table2 = embed_weight.reshape(_VOCAB // 2, 2 * _D)
    gathered = _make_sc_hash_gather(tok.shape[0])(tok, tokp, table2)
    out = _tc_project(gathered, proj_weight.T, scale)
    return out.reshape(b, s, _N_OUT)


# speedup vs baseline: 1.0982x; 1.0982x over previous
import functools

import jax
import jax.numpy as jnp
from jax import lax
from jax.experimental import pallas as pl
from jax.experimental.pallas import tpu as pltpu
from jax.experimental.pallas import tpu_sc as plsc

_VOCAB = 1000000
_MOD = _VOCAB - 1
_D = 64
_N_OUT = 1024
_SEQ = 8192

_NC, _NS = 2, 16
_NW = _NC * _NS
_LANES = 16
_K = 32


def _make_sc_hash_gather(n_tok):
    b_per_w = n_tok // _NW
    n_vec = b_per_w // _LANES
    n_round = b_per_w // _K
    mesh = plsc.VectorSubcoreMesh(core_axis_name="c", subcore_axis_name="s")

    @functools.partial(
        pl.kernel,
        out_type=jax.ShapeDtypeStruct((n_tok // 2, 2 * _D), jnp.float32),
        mesh=mesh,
        scratch_types=[
            pltpu.VMEM((b_per_w,), jnp.int32),
            pltpu.VMEM((b_per_w,), jnp.int32),
            pltpu.VMEM((b_per_w,), jnp.int32),
            pltpu.VMEM((8 * _K, _D), jnp.float32),
            pltpu.VMEM((b_per_w // 2, 2 * _D), jnp.float32),
            pltpu.SemaphoreType.DMA,
        ],
    )
    def sc_kernel(tok_hbm, tokp_hbm, table_hbm, out_hbm,
                  tbuf, pbuf, idx, stag, rows, sem):
        wid = lax.axis_index("s") * _NC + lax.axis_index("c")
        base = wid * b_per_w
        srs = jnp.int32(1) - jnp.minimum(lax.rem(base, jnp.int32(_SEQ)),
                                         jnp.int32(1))

        pltpu.sync_copy(tok_hbm.at[pl.ds(base, b_per_w)], tbuf)
        pltpu.sync_copy(tokp_hbm.at[pl.ds(base, b_per_w)], pbuf)

        lanes = lax.iota(jnp.int32, _LANES)
        lane0 = jnp.int32(1) - jnp.minimum(lanes, jnp.int32(1))
        for j in range(n_vec):
            cur = tbuf[pl.ds(_LANES * j, _LANES)]
            prev = pbuf[pl.ds(_LANES * j, _LANES)]
            mixed = jnp.int32(36313) * cur ^ jnp.int32(27191) * prev
            r = lax.rem(mixed, jnp.int32(_MOD))
            r = r - (r >> 31) * jnp.int32(_MOD)
            if j == 0:
                m = lane0 * srs
                r = r + m * (jnp.int32(_MOD) - r)
            idx[pl.ds(_LANES * j, _LANES)] = r

        def round_body(rd, carry):
            tok0 = rd * _K
            idxvs = [idx[pl.ds(tok0 + _LANES * q, _LANES)]
                     for q in range(_K // _LANES)]
            copies = []
            for q in range(_K // _LANES):
                for k in range(_LANES):
                    v = idxvs[q][k]
                    a = pl.multiple_of((v >> 3) * 8, 8)
                    m = q * _LANES + k
                    copies.append(pltpu.async_copy(
                        table_hbm.at[pl.ds(a, 8), :],
                        stag.at[pl.ds(8 * m, 8), :], sem))
            for cp in copies:
                cp.wait()
            for q in range(_K // _LANES):
                for k in range(_LANES):
                    v = idxvs[q][k]
                    m = q * _LANES + k
                    row = 8 * m + (v & 7)
                    i = tok0 + m
                    for t in range(_D // _LANES):
                        rows[i // 2, pl.ds((i % 2) * _D + t * _LANES,
                                           _LANES)] = (
                            stag[row, pl.ds(t * _LANES, _LANES)])
            return carry

        lax.fori_loop(0, n_round, round_body, jnp.int32(0))
        obase = pl.multiple_of(wid * (b_per_w // 2), b_per_w // 2)
        nflush = 8
        fsz = b_per_w // 2 // nflush
        for f in range(nflush):
            pltpu.sync_copy(rows.at[pl.ds(f * fsz, fsz), :],
                            out_hbm.at[pl.ds(obase + f * fsz, fsz)])

    return sc_kernel


_TM = 1024


def _tc_project(gathered, proj_t, scale):
    n_pair = gathered.shape[0]

    def body(scale_ref, g_ref, p_ref, o_ref):
        s = scale_ref[0]
        o_ref[:, 0, :] = (
            jnp.dot(g_ref[:, :_D], p_ref[...],
                    preferred_element_type=jnp.float32) * s
        )
        o_ref[:, 1, :] = (
            jnp.dot(g_ref[:, _D:], p_ref[...],
                    preferred_element_type=jnp.float32) * s
        )

    return pl.pallas_call(
        body,
        grid=(n_pair // _TM,),
        in_specs=[
            pl.BlockSpec(memory_space=pltpu.SMEM),
            pl.BlockSpec((_TM, 2 * _D), lambda i: (i, 0)),
            pl.BlockSpec((_D, _N_OUT), lambda i: (0, 0)),
        ],
        out_specs=pl.BlockSpec((_TM, 2, _N_OUT), lambda i: (i, 0, 0)),
        out_shape=jax.ShapeDtypeStruct((n_pair, 2, _N_OUT), jnp.float32),
    )(scale.reshape(1).astype(jnp.float32), gathered, proj_t)


def kernel(token_ids, embed_weight, proj_weight, scale):
    b, s = token_ids.shape
    tok2d = token_ids.astype(jnp.int32)
    tokp2d = jnp.concatenate([tok2d[:, :1], tok2d[:, :-1]], axis=1)
    tok = tok2d.reshape(-1)
    tokp = tokp2d.reshape(-1)
    gathered = _make_sc_hash_gather(tok.shape[0])(tok, tokp, embed_weight)
    out = _tc_project(gathered, proj_weight.T, scale)
    return out.reshape(b, s, _N_OUT)


# speedup vs baseline: 2.0455x; 1.8626x over previous
import functools

import jax
import jax.numpy as jnp
from jax import lax
from jax.experimental import pallas as pl
from jax.experimental.pallas import tpu as pltpu
from jax.experimental.pallas import tpu_sc as plsc

_VOCAB = 1000000
_MOD = _VOCAB - 1
_D = 64
_N_OUT = 1024
_SEQ = 8192

_NC, _NS = 2, 16
_NW = _NC * _NS
_LANES = 16
_IDX_CHUNK = 128

_CV = 8192
_Q = 262144


def _tc_repack(embed_weight):
    table_t = embed_weight.T
    n_chunks = _Q // _CV

    def body(g0_ref, g1_ref, g2_ref, g3_ref, x_ref):
        for q, g_ref in enumerate((g0_ref, g1_ref, g2_ref, g3_ref)):
            bits = jax.lax.bitcast_convert_type(g_ref[...], jnp.int32)
            rb = bits + jnp.int32(0x7FFF) + ((bits >> 16) & jnp.int32(1))
            lo = (rb[: _D // 2, :] >> 16) & jnp.int32(0xFFFF)
            hi = rb[_D // 2:, :] & jnp.int32(-65536)
            words = lo | hi
            x_ref[:, q * (_D // 2): (q + 1) * (_D // 2)] = words.T

    last_block = (_VOCAB + _CV - 1) // _CV - 1

    specs = [
        pl.BlockSpec((_D, _CV),
                     functools.partial(
                         lambda i, q: (0, jnp.minimum(i + q * (_Q // _CV),
                                                      last_block)),
                         q=q))
        for q in range(4)
    ]
    return pl.pallas_call(
        body,
        grid=(n_chunks,),
        in_specs=specs,
        out_specs=pl.BlockSpec((_CV, 2 * _D), lambda i: (i, 0)),
        out_shape=jax.ShapeDtypeStruct((_Q, 2 * _D), jnp.int32),
    )(table_t, table_t, table_t, table_t)


def _make_sc_hash_gather(n_tok):
    b_per_w = n_tok // _NW
    n_vec = b_per_w // _LANES
    n_chunk = b_per_w // _IDX_CHUNK
    mesh = plsc.VectorSubcoreMesh(core_axis_name="c", subcore_axis_name="s")

    @functools.partial(
        pl.kernel,
        out_type=jax.ShapeDtypeStruct((n_tok, 2 * _D), jnp.int32),
        mesh=mesh,
        scratch_types=[
            pltpu.VMEM((b_per_w,), jnp.int32),
            pltpu.VMEM((b_per_w,), jnp.int32),
            pltpu.VMEM((n_chunk, _IDX_CHUNK), jnp.int32),
            pltpu.VMEM((b_per_w,), jnp.int32),
            pltpu.VMEM((b_per_w // 2, 2 * _D), jnp.int32),
            pltpu.SemaphoreType.DMA,
        ],
    )
    def sc_kernel(tok_hbm, tokp_hbm, table_hbm, out_hbm,
                  tbuf, pbuf, idx, par, rows, sem):
        wid = lax.axis_index("s") * _NC + lax.axis_index("c")
        base = wid * b_per_w
        srs = jnp.int32(1) - jnp.minimum(lax.rem(base, jnp.int32(_SEQ)),
                                         jnp.int32(1))

        pltpu.sync_copy(tok_hbm.at[pl.ds(base, b_per_w)], tbuf)
        pltpu.sync_copy(tokp_hbm.at[pl.ds(base, b_per_w)], pbuf)

        lanes = lax.iota(jnp.int32, _LANES)
        lane0 = jnp.int32(1) - jnp.minimum(lanes, jnp.int32(1))
        for j in range(n_vec):
            cur = tbuf[pl.ds(_LANES * j, _LANES)]
            prev = pbuf[pl.ds(_LANES * j, _LANES)]
            mixed = jnp.int32(36313) * cur ^ jnp.int32(27191) * prev
            r = lax.rem(mixed, jnp.int32(_MOD))
            r = r - (r >> 31) * jnp.int32(_MOD)
            if j == 0:
                m = lane0 * srs
                r = r + m * (jnp.int32(_MOD) - r)
            idx[j // 8, pl.ds((j % 8) * _LANES, _LANES)] = r & jnp.int32(_Q - 1)
            par[pl.ds(_LANES * j, _LANES)] = (r >> 18) * jnp.int32(2 * _D // 4)

        qw = 2 * _D // 4
        zeros = jnp.zeros((_LANES,), jnp.int32)
        half = b_per_w // 2
        for h in range(2):
            copies = [
                pltpu.async_copy(
                    table_hbm.at[idx.at[h * (n_chunk // 2) + c]],
                    rows.at[pl.ds(c * _IDX_CHUNK, _IDX_CHUNK), :],
                    sem,
                )
                for c in range(n_chunk // 2)
            ]
            for cp in copies:
                cp.wait()

            def zero_body(j, carry):
                pv = par[pl.ds(h * half + j * _LANES, _LANES)]
                for k in range(_LANES):
                    keep = pl.multiple_of(pv[k], qw)
                    i = j * _LANES + k
                    kept = [rows[i, pl.ds(keep + t * _LANES, _LANES)]
                            for t in range(qw // _LANES)]
                    for t in range(2 * _D // _LANES):
                        rows[i, pl.ds(t * _LANES, _LANES)] = zeros
                    for t in range(qw // _LANES):
                        rows[i, pl.ds(keep + t * _LANES, _LANES)] = kept[t]
                return carry

            lax.fori_loop(0, half // _LANES, zero_body, jnp.int32(0))
            pltpu.sync_copy(rows, out_hbm.at[pl.ds(base + h * half, half)])

    return sc_kernel


_TM = 2048


def _tc_project(gathered, proj_t, scale):
    n_tok = gathered.shape[0]

    def body(scale_ref, g_ref, p_ref, o_ref):
        g = g_ref[...]
        f_lo = jax.lax.bitcast_convert_type(g << 16, jnp.float32)
        f_hi = jax.lax.bitcast_convert_type(
            g & jnp.int32(-65536), jnp.float32)
        q = _D // 2
        lhs_lo = (f_lo[:, :q] + f_lo[:, q:2 * q]
                  + f_lo[:, 2 * q:3 * q] + f_lo[:, 3 * q:])
        lhs_hi = (f_hi[:, :q] + f_hi[:, q:2 * q]
                  + f_hi[:, 2 * q:3 * q] + f_hi[:, 3 * q:])
        lhs = jnp.concatenate([lhs_lo, lhs_hi], axis=1)
        o_ref[...] = (
            jnp.dot(lhs, p_ref[...], preferred_element_type=jnp.float32)
            * scale_ref[0]
        )

    return pl.pallas_call(
        body,
        grid=(n_tok // _TM,),
        in_specs=[
            pl.BlockSpec(memory_space=pltpu.SMEM),
            pl.BlockSpec((_TM, 2 * _D), lambda i: (i, 0)),
            pl.BlockSpec((_D, _N_OUT), lambda i: (0, 0)),
        ],
        out_specs=pl.BlockSpec((_TM, _N_OUT), lambda i: (i, 0)),
        out_shape=jax.ShapeDtypeStruct((n_tok, _N_OUT), jnp.float32),
    )(scale.reshape(1).astype(jnp.float32), gathered, proj_t)


def kernel(token_ids, embed_weight, proj_weight, scale):
    b, s = token_ids.shape
    tok2d = token_ids.astype(jnp.int32)
    tokp2d = jnp.concatenate([tok2d[:, :1], tok2d[:, :-1]], axis=1)
    tok = tok2d.reshape(-1)
    tokp = tokp2d.reshape(-1)
    table2 = _tc_repack(embed_weight)
    gathered = _make_sc_hash_gather(tok.shape[0])(tok, tokp, table2)
    out = _tc_project(gathered, proj_weight.T, scale)
    return out.reshape(b, s, _N_OUT)
